# baseline (speedup 1.0000x reference)
import functools

import jax
import jax.numpy as jnp
from jax import lax
from jax.experimental import pallas as pl
from jax.experimental.pallas import tpu as pltpu
from jax.experimental.pallas import tpu_sc as plsc

D_MODEL = 1024
MAX_LEN = 2048
BATCH = 4
LANES = 16
N_WORKERS = 32
P_PER_W = MAX_LEN // N_WORKERS
CHUNK = 8
N_PBLK = P_PER_W // CHUNK
N_CHUNKS = N_PBLK * BATCH
N_RB = 4
SCALE = 32.0

_mesh = plsc.VectorSubcoreMesh(core_axis_name="c", subcore_axis_name="s")


@functools.partial(
    pl.kernel,
    mesh=_mesh,
    out_type=jax.ShapeDtypeStruct((BATCH, MAX_LEN, D_MODEL), jnp.float32),
    scratch_types=[
        pltpu.VMEM((BATCH, P_PER_W), jnp.int32),
        pltpu.VMEM((N_RB, CHUNK, D_MODEL), jnp.float32),
        pltpu.VMEM((P_PER_W, D_MODEL), jnp.float32),
        pltpu.SemaphoreType.DMA,
        [pltpu.SemaphoreType.DMA] * N_RB,
        [pltpu.SemaphoreType.DMA] * N_RB,
    ],
)
def _emb_pe_kernel(x_hbm, table_hbm, pe_hbm, out_hbm,
                   idx_v, rows_v, pe_v, sem_pe, sem_g, sem_o):
    cid = lax.axis_index("c")
    sid = lax.axis_index("s")
    wid = sid * 2 + cid
    p0 = pl.multiple_of(wid * P_PER_W, P_PER_W)

    pe_cp = pltpu.async_copy(pe_hbm.at[pl.ds(p0, P_PER_W)], pe_v, sem_pe)
    for qb in range(BATCH):
        pltpu.sync_copy(x_hbm.at[qb, pl.ds(p0, P_PER_W)], idx_v.at[qb])

    def issue_gather(pblk, qb, b):
        off = pl.multiple_of(pblk * CHUNK, CHUNK)
        pltpu.async_copy(table_hbm.at[idx_v.at[qb, pl.ds(off, CHUNK)]],
                         rows_v.at[b], sem_g[b])

    def issue_out(pblk, qb, b):
        off = pl.multiple_of(p0 + pblk * CHUNK, CHUNK)
        pltpu.async_copy(rows_v.at[b], out_hbm.at[qb, pl.ds(off, CHUNK)],
                         sem_o[b])

    def wait_gather(b):
        pltpu.make_async_copy(table_hbm.at[pl.ds(0, CHUNK)],
                              rows_v.at[b], sem_g[b]).wait()

    def wait_out(b):
        pltpu.make_async_copy(rows_v.at[b], out_hbm.at[0, pl.ds(0, CHUNK)],
                              sem_o[b]).wait()

    issue_gather(0, 0, 0)
    issue_gather(0, 1, 1)
    pe_cp.wait()

    def quad_body(i, carry):
        for b in range(N_RB):
            c = i * N_RB + b
            pblk = i + (b + 2) // 4
            qb2 = (b + 2) % 4

            @pl.when((c >= 2) & (c < N_CHUNKS - 2))
            def _():
                wait_out((b + 2) % N_RB)

            @pl.when(c < N_CHUNKS - 2)
            def _():
                issue_gather(pblk, qb2, (b + 2) % N_RB)

            wait_gather(b)

            def row_body(j, c2):
                pr = i * CHUNK + j
                for l in range(D_MODEL // LANES):
                    sl = pl.ds(l * LANES, LANES)
                    rows_v[b, j, sl] = rows_v[b, j, sl] * SCALE + pe_v[pr, sl]
                return c2

            lax.fori_loop(0, CHUNK, row_body, 0, unroll=False)
            issue_out(i, b, b)
        return carry

    lax.fori_loop(0, N_PBLK, quad_body, 0, unroll=False)

    for b in range(N_RB):
        wait_out(b)


def kernel(x, table, pe):
    xf = x.reshape(BATCH, MAX_LEN).astype(jnp.int32)
    pef = pe.reshape(MAX_LEN, D_MODEL)
    return _emb_pe_kernel(xf, table, pef)


# speedup vs baseline: 1.5013x; 1.5013x over previous
import functools

import jax
import jax.numpy as jnp
from jax import lax
from jax.experimental import pallas as pl
from jax.experimental.pallas import tpu as pltpu
from jax.experimental.pallas import tpu_sc as plsc

D_MODEL = 1024
MAX_LEN = 2048
BATCH = 4
LANES = 16
N_WORKERS = 32
P_PER_W = MAX_LEN // N_WORKERS
CHUNK = 16
N_QUADS = P_PER_W // CHUNK
N_CHUNKS = N_QUADS * BATCH
N_RB = 4
SCALE = 32.0

_mesh = plsc.VectorSubcoreMesh(core_axis_name="c", subcore_axis_name="s")


@functools.partial(
    pl.kernel,
    mesh=_mesh,
    out_type=jax.ShapeDtypeStruct((BATCH, MAX_LEN, D_MODEL), jnp.float32),
    scratch_types=[
        pltpu.VMEM((BATCH, P_PER_W), jnp.int32),
        pltpu.VMEM((N_RB, CHUNK, D_MODEL), jnp.float32),
        pltpu.VMEM((2, CHUNK, D_MODEL), jnp.float32),
        [pltpu.SemaphoreType.DMA] * 2,
        [pltpu.SemaphoreType.DMA] * N_RB,
        [pltpu.SemaphoreType.DMA] * N_RB,
    ],
)
def _emb_pe_kernel(x_hbm, table_hbm, pe_hbm, out_hbm,
                   idx_v, rows_v, pe_v, sem_pe, sem_g, sem_o):
    cid = lax.axis_index("c")
    sid = lax.axis_index("s")
    wid = sid * 2 + cid
    p0 = pl.multiple_of(wid * P_PER_W, P_PER_W)

    for qb in range(BATCH):
        pltpu.sync_copy(x_hbm.at[qb, pl.ds(p0, P_PER_W)], idx_v.at[qb])

    def issue_pe(q, s):
        off = pl.multiple_of(p0 + q * CHUNK, CHUNK)
        pltpu.async_copy(pe_hbm.at[pl.ds(off, CHUNK)], pe_v.at[s], sem_pe[s])

    def wait_pe(s):
        pltpu.make_async_copy(pe_hbm.at[pl.ds(0, CHUNK)], pe_v.at[s],
                              sem_pe[s]).wait()

    def issue_gather(pblk, qb, b):
        off = pl.multiple_of(pblk * CHUNK, CHUNK)
        pltpu.async_copy(table_hbm.at[idx_v.at[qb, pl.ds(off, CHUNK)]],
                         rows_v.at[b], sem_g[b])

    def issue_out(pblk, qb, b):
        off = pl.multiple_of(p0 + pblk * CHUNK, CHUNK)
        pltpu.async_copy(rows_v.at[b], out_hbm.at[qb, pl.ds(off, CHUNK)],
                         sem_o[b])

    def wait_gather(b):
        pltpu.make_async_copy(table_hbm.at[pl.ds(0, CHUNK)],
                              rows_v.at[b], sem_g[b]).wait()

    def wait_out(b):
        pltpu.make_async_copy(rows_v.at[b], out_hbm.at[0, pl.ds(0, CHUNK)],
                              sem_o[b]).wait()

    issue_pe(0, 0)
    issue_gather(0, 0, 0)
    issue_gather(0, 1, 1)

    def pair_body(ip, carry):
        for qq in range(2):
            q = ip * 2 + qq

            @pl.when(q < N_QUADS - 1)
            def _():
                issue_pe(q + 1, 1 - qq)

            wait_pe(qq)

            for b in range(N_RB):
                c = q * N_RB + b
                pblk2 = q + (b + 2) // 4
                qb2 = (b + 2) % 4

                @pl.when((c >= 2) & (c < N_CHUNKS - 2))
                def _():
                    wait_out((b + 2) % N_RB)

                @pl.when(c < N_CHUNKS - 2)
                def _():
                    issue_gather(pblk2, qb2, (b + 2) % N_RB)

                wait_gather(b)

                def row_body(j, c2):
                    for l in range(D_MODEL // LANES):
                        sl = pl.ds(l * LANES, LANES)
                        rows_v[b, j, sl] = (rows_v[b, j, sl] * SCALE
                                            + pe_v[qq, j, sl])
                    return c2

                lax.fori_loop(0, CHUNK, row_body, 0, unroll=False)
                issue_out(q, b, b)
        return carry

    lax.fori_loop(0, N_QUADS // 2, pair_body, 0, unroll=False)

    for b in range(N_RB):
        wait_out(b)


def kernel(x, table, pe):
    xf = x.reshape(BATCH, MAX_LEN).astype(jnp.int32)
    pef = pe.reshape(MAX_LEN, D_MODEL)
    return _emb_pe_kernel(xf, table, pef)


# speedup vs baseline: 2.2910x; 1.5261x over previous
import functools

import jax
import jax.numpy as jnp
from jax import lax
from jax.experimental import pallas as pl
from jax.experimental.pallas import tpu as pltpu
from jax.experimental.pallas import tpu_sc as plsc

D_MODEL = 1024
MAX_LEN = 2048
BATCH = 4
LANES = 16
N_WORKERS = 32
P_PER_W = MAX_LEN // N_WORKERS
CHUNK = 16
N_QUADS = P_PER_W // CHUNK
N_CHUNKS = N_QUADS * BATCH
N_RB = 4
GROUPS = D_MODEL // LANES
SCALE = 32.0

_mesh = plsc.VectorSubcoreMesh(core_axis_name="c", subcore_axis_name="s")


@functools.partial(
    pl.kernel,
    mesh=_mesh,
    out_type=jax.ShapeDtypeStruct((BATCH, MAX_LEN, D_MODEL), jnp.float32),
    scratch_types=[
        pltpu.VMEM((BATCH, P_PER_W), jnp.int32),
        pltpu.VMEM((N_RB, CHUNK, D_MODEL), jnp.float32),
        pltpu.VMEM((2, CHUNK, D_MODEL), jnp.float32),
        [pltpu.SemaphoreType.DMA] * 2,
        [pltpu.SemaphoreType.DMA] * N_RB,
        [pltpu.SemaphoreType.DMA] * N_RB,
    ],
)
def _emb_pe_kernel(x_hbm, table_hbm, pe_hbm, out_hbm,
                   idx_v, rows_v, pe_v, sem_pe, sem_g, sem_o):
    cid = lax.axis_index("c")
    sid = lax.axis_index("s")
    wid = sid * 2 + cid
    p0 = pl.multiple_of(wid * P_PER_W, P_PER_W)

    for qb in range(BATCH):
        pltpu.sync_copy(x_hbm.at[qb, pl.ds(p0, P_PER_W)], idx_v.at[qb])

    def issue_pe(q, s):
        off = pl.multiple_of(p0 + q * CHUNK, CHUNK)
        pltpu.async_copy(pe_hbm.at[pl.ds(off, CHUNK)], pe_v.at[s], sem_pe[s])

    def wait_pe(s):
        pltpu.make_async_copy(pe_hbm.at[pl.ds(0, CHUNK)], pe_v.at[s],
                              sem_pe[s]).wait()

    def issue_gather(pblk, qb, b):
        off = pl.multiple_of(pblk * CHUNK, CHUNK)
        pltpu.async_copy(table_hbm.at[idx_v.at[qb, pl.ds(off, CHUNK)]],
                         rows_v.at[b], sem_g[b])

    def issue_out(pblk, qb, b):
        off = pl.multiple_of(p0 + pblk * CHUNK, CHUNK)
        pltpu.async_copy(rows_v.at[b], out_hbm.at[qb, pl.ds(off, CHUNK)],
                         sem_o[b])

    def wait_gather(b):
        pltpu.make_async_copy(table_hbm.at[pl.ds(0, CHUNK)],
                              rows_v.at[b], sem_g[b]).wait()

    def wait_out(b):
        pltpu.make_async_copy(rows_v.at[b], out_hbm.at[0, pl.ds(0, CHUNK)],
                              sem_o[b]).wait()

    issue_pe(0, 0)
    issue_gather(0, 0, 0)
    issue_gather(0, 1, 1)

    def pair_body(ip, carry):
        for qq in range(2):
            q = ip * 2 + qq

            @pl.when(q < N_QUADS - 1)
            def _():
                issue_pe(q + 1, 1 - qq)

            wait_pe(qq)

            for b in range(N_RB):
                c = q * N_RB + b
                pblk2 = q + (b + 2) // 4
                qb2 = (b + 2) % 4

                @pl.when((c >= 2) & (c < N_CHUNKS - 2))
                def _():
                    wait_out((b + 2) % N_RB)

                @pl.when(c < N_CHUNKS - 2)
                def _():
                    issue_gather(pblk2, qb2, (b + 2) % N_RB)

                wait_gather(b)

                @plsc.parallel_loop(0, CHUNK * GROUPS, step=1, unroll=4)
                def _(t):
                    j = t // GROUPS
                    l = t - j * GROUPS
                    sl = pl.ds(l * LANES, LANES)
                    rows_v[b, j, sl] = (rows_v[b, j, sl] * SCALE
                                        + pe_v[qq, j, sl])

                issue_out(q, b, b)
        return carry

    lax.fori_loop(0, N_QUADS // 2, pair_body, 0, unroll=False)

    for b in range(N_RB):
        wait_out(b)


def kernel(x, table, pe):
    xf = x.reshape(BATCH, MAX_LEN).astype(jnp.int32)
    pef = pe.reshape(MAX_LEN, D_MODEL)
    return _emb_pe_kernel(xf, table, pef)
